# baseline (speedup 1.0000x reference)
import functools

import jax
import jax.numpy as jnp
from jax import lax
from jax.experimental import pallas as pl
from jax.experimental.pallas import tpu as pltpu
from jax.experimental.pallas import tpu_sc as plsc

NUM_CORES = 2
NUM_SUBCORES = 16
NUM_WORKERS = NUM_CORES * NUM_SUBCORES
CHUNK_B = 8


@functools.lru_cache(maxsize=None)
def _make_gather(n_b, h, d):
    b_per_w = n_b // NUM_WORKERS
    n_per_w = b_per_w * h
    chunk = CHUNK_B * h
    n_chunks = b_per_w // CHUNK_B
    assert b_per_w % (2 * CHUNK_B) == 0
    mesh = plsc.VectorSubcoreMesh(
        core_axis_name="c", subcore_axis_name="s",
        num_cores=NUM_CORES, num_subcores=NUM_SUBCORES)

    @functools.partial(
        pl.kernel,
        mesh=mesh,
        out_type=jax.ShapeDtypeStruct((n_b * h, d), jnp.float32),
        scratch_types=[
            pltpu.VMEM((n_per_w,), jnp.int32),
            [pltpu.VMEM((chunk, d), jnp.float32) for _ in range(2)],
            [pltpu.SemaphoreType.DMA for _ in range(2)],
            [pltpu.SemaphoreType.DMA for _ in range(2)],
        ],
        compiler_params=pltpu.CompilerParams(use_tc_tiling_on_sc=False),
    )
    def gather_kernel(table_hbm, idx_hbm, out_hbm, idx_v, rows, gsem, ssem):
        wid = lax.axis_index("s") * NUM_CORES + lax.axis_index("c")
        base = wid * n_per_w

        def start_gather(t, rb):
            pltpu.async_copy(
                table_hbm.at[idx_v.at[pl.ds(t * chunk, chunk)]],
                rows[rb], gsem[rb])

        def wait_gather(t, rb):
            pltpu.make_async_copy(
                table_hbm.at[idx_v.at[pl.ds(t * chunk, chunk)]],
                rows[rb], gsem[rb]).wait()

        def start_store(t, rb):
            pltpu.async_copy(
                rows[rb], out_hbm.at[pl.ds(base + t * chunk, chunk), :],
                ssem[rb])

        def wait_store(t, rb):
            pltpu.make_async_copy(
                rows[rb], out_hbm.at[pl.ds(base + t * chunk, chunk), :],
                ssem[rb]).wait()

        pltpu.sync_copy(idx_hbm.at[pl.ds(base, n_per_w)], idx_v)

        start_gather(0, 0)
        start_gather(1, 1)

        def pair_body(p, carry):
            for half in range(2):
                t = p * 2 + half
                wait_gather(t, half)
                start_store(t, half)
                wait_store(t, half)

                @pl.when(t + 2 < n_chunks)
                def _():
                    start_gather(t + 2, half)
            return carry

        lax.fori_loop(0, n_chunks // 2, pair_body, 0)

    return gather_kernel


def kernel(label, bb, table):
    del bb
    b, h = label.shape
    idx = label.reshape(b * h).astype(jnp.int32)
    out = _make_gather(b, h, table.shape[1])(table, idx)
    return out.reshape(b, h, table.shape[1])
